# Initial kernel scaffold; baseline (speedup 1.0000x reference)
import functools

import jax
import jax.numpy as jnp
from jax import lax
from jax.experimental import pallas as pl
from jax.experimental.pallas import tpu as pltpu
from jax.experimental.pallas import tpu_sc as plsc

B, S, D = 1024, 20, 64
NNEG = 100
BS = B * S
W = 112
NC, NS = 2, 16
NW = NC * NS
PPW = BS // NW
P = 4
CH = PPW // P
NBLK = W // 16


def _sc_body(table, idx_hbm, seq_hbm, out_hbm, idx_v, rows_v, qs_v, sc_v, sem):
    cid = lax.axis_index("c")
    sid = lax.axis_index("s")
    wid = sid * NC + cid
    lane = lax.iota(jnp.int32, 16)

    def chunk_body(c, _):
        pair0 = wid * PPW + c * P
        pltpu.sync_copy(idx_hbm.at[pl.ds(pair0 * W, P * W)], idx_v)
        pltpu.async_copy(table.at[idx_v], rows_v, sem).wait()
        pltpu.sync_copy(seq_hbm.at[pl.ds(pair0, P)], qs_v)

        def pair_body(p, _):
            pvec = jnp.full((16,), p, dtype=jnp.int32)
            rbase = p * W
            row_idx = [rbase + j * 16 + lane for j in range(NBLK)]
            accs = [jnp.zeros((16,), jnp.float32) for _ in range(NBLK)]
            for d in range(D):
                dvec = jnp.full((16,), d, dtype=jnp.int32)
                qd = plsc.load_gather(qs_v, [pvec, dvec])
                for j in range(NBLK):
                    g = plsc.load_gather(rows_v, [row_idx[j], dvec])
                    accs[j] = accs[j] + g * qd
            for j in range(NBLK):
                sc_v[pl.ds(p * W + j * 16, 16)] = accs[j]
            return 0

        lax.fori_loop(0, P, pair_body, 0)
        pltpu.sync_copy(sc_v, out_hbm.at[pl.ds(pair0 * W, P * W)])
        return 0

    lax.fori_loop(0, CH, chunk_body, 0)


_sc_scores = pl.kernel(
    _sc_body,
    out_type=jax.ShapeDtypeStruct((BS * W,), jnp.float32),
    mesh=plsc.VectorSubcoreMesh(
        core_axis_name="c", subcore_axis_name="s", num_cores=NC, num_subcores=NS
    ),
    scratch_types=[
        pltpu.VMEM((P * W,), jnp.int32),
        pltpu.VMEM((P * W, D), jnp.float32),
        pltpu.VMEM((P, D), jnp.float32),
        pltpu.VMEM((P * W,), jnp.float32),
        pltpu.SemaphoreType.DMA,
    ],
)


_R = 2048


def _tc_loss_body(sc_ref, mask_ref, out_ref, acc_ref):
    i = pl.program_id(0)

    @pl.when(i == 0)
    def _():
        acc_ref[0] = 0.0
        acc_ref[1] = 0.0

    s = sc_ref[...]
    m = mask_ref[...]
    pos = s[:, NNEG:NNEG + 1]
    neg = s[:, :NNEG]
    x = pos - neg
    sig = 1.0 / (1.0 + jnp.exp(-x))
    loss = -jnp.log(sig + 1e-8)
    acc_ref[0] += jnp.sum(loss * m)
    acc_ref[1] += jnp.sum(m)

    @pl.when(i == pl.num_programs(0) - 1)
    def _():
        out_ref[0, 0] = acc_ref[0] / (acc_ref[1] * NNEG)


_tc_loss = pl.pallas_call(
    _tc_loss_body,
    grid=(BS // _R,),
    in_specs=[
        pl.BlockSpec((_R, W), lambda i: (i, 0)),
        pl.BlockSpec((_R, 1), lambda i: (i, 0)),
    ],
    out_specs=pl.BlockSpec((1, 1), lambda i: (0, 0), memory_space=pltpu.SMEM),
    out_shape=jax.ShapeDtypeStruct((1, 1), jnp.float32),
    scratch_shapes=[pltpu.SMEM((2,), jnp.float32)],
)


@jax.jit
def kernel(seq_embs, target_seq, mask, item_emb_table, neg_items):
    seq2 = seq_embs.reshape(BS, D)
    idx_all = jnp.concatenate(
        [
            neg_items.reshape(BS, NNEG),
            target_seq.reshape(BS, 1),
            jnp.zeros((BS, W - NNEG - 1), dtype=jnp.int32),
        ],
        axis=1,
    ).reshape(BS * W)
    scores = _sc_scores(item_emb_table, idx_all, seq2).reshape(BS, W)
    out = _tc_loss(scores, mask.reshape(BS, 1))
    return out[0, 0]


# speedup vs baseline: 2.2358x; 2.2358x over previous
import functools

import jax
import jax.numpy as jnp
from jax import lax
from jax.experimental import pallas as pl
from jax.experimental.pallas import tpu as pltpu
from jax.experimental.pallas import tpu_sc as plsc

B, S, D = 1024, 20, 64
NNEG = 100
BS = B * S
W = 112
NC, NS = 2, 16
NW = NC * NS
PPW = BS // NW
P = 4
CH = PPW // P
NBLK = W // 16


def _sc_body(table, idx_hbm, seq_hbm, out_hbm, idx_v, rows_v, qs_v, sc_v, sem):
    cid = lax.axis_index("c")
    sid = lax.axis_index("s")
    wid = sid * NC + cid
    lane = lax.iota(jnp.int32, 16)

    def chunk_body(c, _):
        pair0 = wid * PPW + c * P
        pltpu.sync_copy(idx_hbm.at[pl.ds(pair0 * W, P * W)], idx_v)
        pltpu.async_copy(table.at[idx_v], rows_v, sem).wait()
        pltpu.sync_copy(seq_hbm.at[pl.ds(pair0, P)], qs_v)

        def pair_body(p, _):
            pvec = jnp.full((16,), p, dtype=jnp.int32)
            rbase = p * W
            row_idx = [rbase + j * 16 + lane for j in range(NBLK)]
            accs = [jnp.zeros((16,), jnp.float32) for _ in range(NBLK)]
            for d in range(D):
                dvec = jnp.full((16,), d, dtype=jnp.int32)
                qd = plsc.load_gather(qs_v, [pvec, dvec])
                for j in range(NBLK):
                    g = plsc.load_gather(rows_v, [row_idx[j], dvec])
                    accs[j] = accs[j] + g * qd
            for j in range(NBLK):
                sc_v[pl.ds(p * W + j * 16, 16)] = accs[j]
            return 0

        lax.fori_loop(0, P, pair_body, 0)
        pltpu.sync_copy(sc_v, out_hbm.at[pl.ds(pair0 * W, P * W)])
        return 0

    lax.fori_loop(0, CH, chunk_body, 0)


@functools.cache
def _sc_scores():
    return pl.kernel(
        _sc_body,
        out_type=jax.ShapeDtypeStruct((BS * W,), jnp.float32),
        mesh=plsc.VectorSubcoreMesh(
            core_axis_name="c", subcore_axis_name="s", num_cores=NC, num_subcores=NS
        ),
        compiler_params=pltpu.CompilerParams(
            needs_layout_passes=False, use_tc_tiling_on_sc=False
        ),
        scratch_types=[
            pltpu.VMEM((P * W,), jnp.int32),
            pltpu.VMEM((P * W, D), jnp.float32),
            pltpu.VMEM((P, D), jnp.float32),
            pltpu.VMEM((P * W,), jnp.float32),
            pltpu.SemaphoreType.DMA,
        ],
    )


_R = 2048


def _tc_loss_body(sc_ref, mask_ref, out_ref, acc_ref):
    i = pl.program_id(0)

    @pl.when(i == 0)
    def _():
        acc_ref[0] = 0.0
        acc_ref[1] = 0.0

    s = sc_ref[...]
    m = mask_ref[...]
    pos = s[:, NNEG:NNEG + 1]
    neg = s[:, :NNEG]
    x = pos - neg
    sig = 1.0 / (1.0 + jnp.exp(-x))
    loss = -jnp.log(sig + 1e-8)
    acc_ref[0] += jnp.sum(loss * m)
    acc_ref[1] += jnp.sum(m)

    @pl.when(i == pl.num_programs(0) - 1)
    def _():
        out_ref[0, 0] = acc_ref[0] / (acc_ref[1] * NNEG)


@functools.cache
def _tc_loss():
    return pl.pallas_call(
        _tc_loss_body,
        grid=(BS // _R,),
        in_specs=[
            pl.BlockSpec((_R, W), lambda i: (i, 0)),
            pl.BlockSpec((_R, 1), lambda i: (i, 0)),
        ],
        out_specs=pl.BlockSpec((1, 1), lambda i: (0, 0), memory_space=pltpu.SMEM),
        out_shape=jax.ShapeDtypeStruct((1, 1), jnp.float32),
        scratch_shapes=[pltpu.SMEM((2,), jnp.float32)],
    )


@jax.jit
def kernel(seq_embs, target_seq, mask, item_emb_table, neg_items):
    seq2 = seq_embs.reshape(BS, D)
    idx_all = jnp.concatenate(
        [
            neg_items.reshape(BS, NNEG),
            target_seq.reshape(BS, 1),
            jnp.zeros((BS, W - NNEG - 1), dtype=jnp.int32),
        ],
        axis=1,
    ).reshape(BS * W)
    scores = _sc_scores()(item_emb_table, idx_all, seq2).reshape(BS, W)
    out = _tc_loss()(scores, mask.reshape(BS, 1))
    return out[0, 0]


# speedup vs baseline: 3.6281x; 1.6227x over previous
import functools

import jax
import jax.numpy as jnp
from jax import lax
from jax.experimental import pallas as pl
from jax.experimental.pallas import tpu as pltpu
from jax.experimental.pallas import tpu_sc as plsc

B, S, D = 1024, 20, 64
NNEG = 100
BS = B * S
W = 112
NC, NS = 2, 16
NW = NC * NS
PPW = BS // NW
P = 8
CH = PPW // P
NBLK = W // 16
NR = P * NNEG + P


def _sc_body(neg_hbm, tgt_hbm, seq_hbm, table, out_hbm,
             idx0, idx1, rows0, rows1, qs0, qs1, sc0, sc1, sem0, sem1):
    bufs = ((idx0, rows0, qs0, sc0, sem0), (idx1, rows1, qs1, sc1, sem1))
    cid = lax.axis_index("c")
    sid = lax.axis_index("s")
    wid = sid * NC + cid
    lane = lax.iota(jnp.int32, 16)

    def stage(c, buf):
        idx_v, rows_v, qs_v, _, sem = buf
        pair0 = wid * PPW + c * P
        pltpu.sync_copy(neg_hbm.at[pl.ds(pair0 * NNEG, P * NNEG)],
                        idx_v.at[pl.ds(0, P * NNEG)])
        pltpu.sync_copy(tgt_hbm.at[pl.ds(pair0, P)],
                        idx_v.at[pl.ds(P * NNEG, P)])
        pltpu.sync_copy(seq_hbm.at[pl.ds(pair0, P)], qs_v)
        pltpu.async_copy(table.at[idx_v], rows_v, sem)

    def compute(c, buf):
        idx_v, rows_v, qs_v, sc_v, sem = buf
        pair0 = wid * PPW + c * P
        pltpu.make_async_copy(table.at[idx_v], rows_v, sem).wait()

        def pair_body(p, _):
            pvec = jnp.full((16,), p, dtype=jnp.int32)
            row_idx = [p * NNEG + j * 16 + lane for j in range(NBLK - 1)]
            row_idx.append(
                jnp.where(lane < 4, p * NNEG + 96 + lane, P * NNEG + p))
            accs = [jnp.zeros((16,), jnp.float32) for _ in range(NBLK)]
            for d in range(D):
                dvec = jnp.full((16,), d, dtype=jnp.int32)
                qd = plsc.load_gather(qs_v, [pvec, dvec])
                for j in range(NBLK):
                    g = plsc.load_gather(rows_v, [row_idx[j], dvec])
                    accs[j] = accs[j] + g * qd
            for j in range(NBLK):
                sc_v[pl.ds(p * W + j * 16, 16)] = accs[j]
            return 0

        lax.fori_loop(0, P, pair_body, 0)
        pltpu.sync_copy(sc_v, out_hbm.at[pl.ds(pair0 * W, P * W)])

    stage(0, bufs[0])

    def outer(c2, _):
        for b in range(2):
            c = c2 * 2 + b

            @pl.when(c + 1 < CH)
            def _():
                stage(c + 1, bufs[1 - b])

            compute(c, bufs[b])
        return 0

    lax.fori_loop(0, CH // 2, outer, 0)


@functools.cache
def _sc_scores():
    return pl.kernel(
        _sc_body,
        out_type=jax.ShapeDtypeStruct((BS * W,), jnp.float32),
        mesh=plsc.VectorSubcoreMesh(
            core_axis_name="c", subcore_axis_name="s", num_cores=NC, num_subcores=NS
        ),
        compiler_params=pltpu.CompilerParams(
            needs_layout_passes=False, use_tc_tiling_on_sc=False
        ),
        scratch_types=[
            pltpu.VMEM((NR,), jnp.int32),
            pltpu.VMEM((NR,), jnp.int32),
            pltpu.VMEM((NR, D), jnp.float32),
            pltpu.VMEM((NR, D), jnp.float32),
            pltpu.VMEM((P, D), jnp.float32),
            pltpu.VMEM((P, D), jnp.float32),
            pltpu.VMEM((P * W,), jnp.float32),
            pltpu.VMEM((P * W,), jnp.float32),
            pltpu.SemaphoreType.DMA,
            pltpu.SemaphoreType.DMA,
        ],
    )


_R = 2048


def _tc_loss_body(sc_ref, mask_ref, out_ref, acc_ref):
    i = pl.program_id(0)

    @pl.when(i == 0)
    def _():
        acc_ref[0] = 0.0
        acc_ref[1] = 0.0

    s = sc_ref[...]
    m = mask_ref[...]
    pos = s[:, NNEG:NNEG + 1]
    neg = s[:, :NNEG]
    x = pos - neg
    sig = 1.0 / (1.0 + jnp.exp(-x))
    loss = -jnp.log(sig + 1e-8)
    acc_ref[0] += jnp.sum(loss * m)
    acc_ref[1] += jnp.sum(m)

    @pl.when(i == pl.num_programs(0) - 1)
    def _():
        out_ref[0, 0] = acc_ref[0] / (acc_ref[1] * NNEG)


@functools.cache
def _tc_loss():
    return pl.pallas_call(
        _tc_loss_body,
        grid=(BS // _R,),
        in_specs=[
            pl.BlockSpec((_R, W), lambda i: (i, 0)),
            pl.BlockSpec((_R, 1), lambda i: (i, 0)),
        ],
        out_specs=pl.BlockSpec((1, 1), lambda i: (0, 0), memory_space=pltpu.SMEM),
        out_shape=jax.ShapeDtypeStruct((1, 1), jnp.float32),
        scratch_shapes=[pltpu.SMEM((2,), jnp.float32)],
    )


@jax.jit
def kernel(seq_embs, target_seq, mask, item_emb_table, neg_items):
    scores = _sc_scores()(
        neg_items.reshape(BS * NNEG),
        target_seq.reshape(BS),
        seq_embs.reshape(BS, D),
        item_emb_table,
    ).reshape(BS, W)
    out = _tc_loss()(scores, mask.reshape(BS, 1))
    return out[0, 0]


# speedup vs baseline: 3.7528x; 1.0344x over previous
import functools

import jax
import jax.numpy as jnp
from jax import lax
from jax.experimental import pallas as pl
from jax.experimental.pallas import tpu as pltpu
from jax.experimental.pallas import tpu_sc as plsc

B, S, D = 1024, 20, 64
NNEG = 100
BS = B * S
W = 112
NC, NS = 2, 16
NW = NC * NS
PPW = BS // NW
P = 8
CH = PPW // P
NBLK = W // 16
NR = P * NNEG + P


def _sc_body(neg_hbm, tgt_hbm, seq_hbm, table, out_hbm,
             idx0, idx1, rows0, rows1, qs0, qs1, sc0, sc1, sem0, sem1):
    bufs = ((idx0, rows0, qs0, sc0, sem0), (idx1, rows1, qs1, sc1, sem1))
    cid = lax.axis_index("c")
    sid = lax.axis_index("s")
    wid = sid * NC + cid
    lane = lax.iota(jnp.int32, 16)

    def stage(c, buf):
        idx_v, rows_v, qs_v, _, sem = buf
        pair0 = wid * PPW + c * P
        pltpu.sync_copy(neg_hbm.at[pl.ds(pair0 * NNEG, P * NNEG)],
                        idx_v.at[pl.ds(0, P * NNEG)])
        pltpu.sync_copy(tgt_hbm.at[pl.ds(pair0, P)],
                        idx_v.at[pl.ds(P * NNEG, P)])
        pltpu.sync_copy(seq_hbm.at[pl.ds(pair0, P)], qs_v)
        pltpu.async_copy(table.at[idx_v], rows_v, sem)

    def compute(c, buf):
        idx_v, rows_v, qs_v, sc_v, sem = buf
        pair0 = wid * PPW + c * P
        pltpu.make_async_copy(table.at[idx_v], rows_v, sem).wait()

        def pair_body(p, _):
            pvec = jnp.full((16,), p, dtype=jnp.int32)
            row_idx = [p * NNEG + j * 16 + lane for j in range(NBLK - 1)]
            row_idx.append(
                jnp.where(lane < 4, p * NNEG + 96 + lane, P * NNEG + p))

            def d_body(i, accs):
                accs = list(accs)
                for k in range(4):
                    d = i * 4 + k
                    dvec = jnp.full((16,), d, dtype=jnp.int32)
                    qd = plsc.load_gather(qs_v, [pvec, dvec])
                    for j in range(NBLK):
                        g = plsc.load_gather(rows_v, [row_idx[j], dvec])
                        accs[j] = accs[j] + g * qd
                return tuple(accs)

            accs = lax.fori_loop(
                0, D // 4, d_body,
                tuple(jnp.zeros((16,), jnp.float32) for _ in range(NBLK)))
            for j in range(NBLK):
                sc_v[pl.ds(p * W + j * 16, 16)] = accs[j]
            return 0

        lax.fori_loop(0, P, pair_body, 0)
        pltpu.sync_copy(sc_v, out_hbm.at[pl.ds(pair0 * W, P * W)])

    stage(0, bufs[0])

    def outer(c2, _):
        for b in range(2):
            c = c2 * 2 + b

            @pl.when(c + 1 < CH)
            def _():
                stage(c + 1, bufs[1 - b])

            compute(c, bufs[b])
        return 0

    lax.fori_loop(0, CH // 2, outer, 0)


@functools.cache
def _sc_scores():
    return pl.kernel(
        _sc_body,
        out_type=jax.ShapeDtypeStruct((BS * W,), jnp.float32),
        mesh=plsc.VectorSubcoreMesh(
            core_axis_name="c", subcore_axis_name="s", num_cores=NC, num_subcores=NS
        ),
        compiler_params=pltpu.CompilerParams(
            needs_layout_passes=False, use_tc_tiling_on_sc=False
        ),
        scratch_types=[
            pltpu.VMEM((NR,), jnp.int32),
            pltpu.VMEM((NR,), jnp.int32),
            pltpu.VMEM((NR, D), jnp.float32),
            pltpu.VMEM((NR, D), jnp.float32),
            pltpu.VMEM((P, D), jnp.float32),
            pltpu.VMEM((P, D), jnp.float32),
            pltpu.VMEM((P * W,), jnp.float32),
            pltpu.VMEM((P * W,), jnp.float32),
            pltpu.SemaphoreType.DMA,
            pltpu.SemaphoreType.DMA,
        ],
    )


_R = 2048


def _tc_loss_body(sc_ref, mask_ref, out_ref, acc_ref):
    i = pl.program_id(0)

    @pl.when(i == 0)
    def _():
        acc_ref[0] = 0.0
        acc_ref[1] = 0.0

    s = sc_ref[...]
    m = mask_ref[...]
    pos = s[:, NNEG:NNEG + 1]
    neg = s[:, :NNEG]
    x = pos - neg
    sig = 1.0 / (1.0 + jnp.exp(-x))
    loss = -jnp.log(sig + 1e-8)
    acc_ref[0] += jnp.sum(loss * m)
    acc_ref[1] += jnp.sum(m)

    @pl.when(i == pl.num_programs(0) - 1)
    def _():
        out_ref[0, 0] = acc_ref[0] / (acc_ref[1] * NNEG)


@functools.cache
def _tc_loss():
    return pl.pallas_call(
        _tc_loss_body,
        grid=(BS // _R,),
        in_specs=[
            pl.BlockSpec((_R, W), lambda i: (i, 0)),
            pl.BlockSpec((_R, 1), lambda i: (i, 0)),
        ],
        out_specs=pl.BlockSpec((1, 1), lambda i: (0, 0), memory_space=pltpu.SMEM),
        out_shape=jax.ShapeDtypeStruct((1, 1), jnp.float32),
        scratch_shapes=[pltpu.SMEM((2,), jnp.float32)],
    )


@jax.jit
def kernel(seq_embs, target_seq, mask, item_emb_table, neg_items):
    scores = _sc_scores()(
        neg_items.reshape(BS * NNEG),
        target_seq.reshape(BS),
        seq_embs.reshape(BS, D),
        item_emb_table,
    ).reshape(BS, W)
    out = _tc_loss()(scores, mask.reshape(BS, 1))
    return out[0, 0]


# speedup vs baseline: 12.0850x; 3.2203x over previous
import functools

import jax
import jax.numpy as jnp
from jax import lax
from jax.experimental import pallas as pl
from jax.experimental.pallas import tpu as pltpu
from jax.experimental.pallas import tpu_sc as plsc

B, S, D = 1024, 20, 64
NNEG = 100
BS = B * S
W = 112
NC, NS = 2, 16
NW = NC * NS
PPW = BS // NW
P = 8
CH = PPW // P
NBLK = W // 16
NR = P * NNEG + P


def _sc_body(neg_hbm, tgt_hbm, seq_hbm, table, out_hbm,
             idx0, idx1, rows0, rows1, qs0, qs1, sc0, sc1, sem0, sem1):
    bufs = ((idx0, rows0, qs0, sc0, sem0), (idx1, rows1, qs1, sc1, sem1))
    cid = lax.axis_index("c")
    sid = lax.axis_index("s")
    wid = sid * NC + cid
    lane = lax.iota(jnp.int32, 16)

    def stage(c, buf):
        idx_v, rows_v, qs_v, _, sem = buf
        pair0 = wid * PPW + c * P
        pltpu.sync_copy(neg_hbm.at[pl.ds(pair0 * NNEG, P * NNEG)],
                        idx_v.at[pl.ds(0, P * NNEG)])
        pltpu.sync_copy(tgt_hbm.at[pl.ds(pair0, P)],
                        idx_v.at[pl.ds(P * NNEG, P)])
        pltpu.sync_copy(seq_hbm.at[pl.ds(pair0, P)], qs_v)
        pltpu.async_copy(table.at[idx_v], rows_v, sem)

    def compute(c, buf):
        idx_v, rows_v, qs_v, sc_v, sem = buf
        pair0 = wid * PPW + c * P
        pltpu.make_async_copy(table.at[idx_v], rows_v, sem).wait()

        lane15 = lane == 15

        def pair_body(p, _):
            q = [qs_v[p, pl.ds(k * 16, 16)] for k in range(D // 16)]

            def score_one(row, pos):
                t0 = (rows_v[row, pl.ds(0, 16)] * q[0]
                      + rows_v[row, pl.ds(16, 16)] * q[1])
                t1 = (rows_v[row, pl.ds(32, 16)] * q[2]
                      + rows_v[row, pl.ds(48, 16)] * q[3])
                c = plsc.cumsum(t0 + t1)
                plsc.store_scatter(
                    sc_v, [jnp.full((16,), pos, dtype=jnp.int32)], c, mask=lane15)

            @plsc.parallel_loop(0, NNEG, step=1, unroll=4)
            def _(n):
                score_one(p * NNEG + n, p * W + n)

            score_one(P * NNEG + p, p * W + NNEG)
            return 0

        lax.fori_loop(0, P, pair_body, 0)
        pltpu.sync_copy(sc_v, out_hbm.at[pl.ds(pair0 * W, P * W)])

    stage(0, bufs[0])

    def outer(c2, _):
        for b in range(2):
            c = c2 * 2 + b

            @pl.when(c + 1 < CH)
            def _():
                stage(c + 1, bufs[1 - b])

            compute(c, bufs[b])
        return 0

    lax.fori_loop(0, CH // 2, outer, 0)


@functools.cache
def _sc_scores():
    return pl.kernel(
        _sc_body,
        out_type=jax.ShapeDtypeStruct((BS * W,), jnp.float32),
        mesh=plsc.VectorSubcoreMesh(
            core_axis_name="c", subcore_axis_name="s", num_cores=NC, num_subcores=NS
        ),
        compiler_params=pltpu.CompilerParams(
            needs_layout_passes=False, use_tc_tiling_on_sc=False
        ),
        scratch_types=[
            pltpu.VMEM((NR,), jnp.int32),
            pltpu.VMEM((NR,), jnp.int32),
            pltpu.VMEM((NR, D), jnp.float32),
            pltpu.VMEM((NR, D), jnp.float32),
            pltpu.VMEM((P, D), jnp.float32),
            pltpu.VMEM((P, D), jnp.float32),
            pltpu.VMEM((P * W,), jnp.float32),
            pltpu.VMEM((P * W,), jnp.float32),
            pltpu.SemaphoreType.DMA,
            pltpu.SemaphoreType.DMA,
        ],
    )


_R = 2048


def _tc_loss_body(sc_ref, mask_ref, out_ref, acc_ref):
    i = pl.program_id(0)

    @pl.when(i == 0)
    def _():
        acc_ref[0] = 0.0
        acc_ref[1] = 0.0

    s = sc_ref[...]
    m = mask_ref[...]
    pos = s[:, NNEG:NNEG + 1]
    neg = s[:, :NNEG]
    x = pos - neg
    sig = 1.0 / (1.0 + jnp.exp(-x))
    loss = -jnp.log(sig + 1e-8)
    acc_ref[0] += jnp.sum(loss * m)
    acc_ref[1] += jnp.sum(m)

    @pl.when(i == pl.num_programs(0) - 1)
    def _():
        out_ref[0, 0] = acc_ref[0] / (acc_ref[1] * NNEG)


@functools.cache
def _tc_loss():
    return pl.pallas_call(
        _tc_loss_body,
        grid=(BS // _R,),
        in_specs=[
            pl.BlockSpec((_R, W), lambda i: (i, 0)),
            pl.BlockSpec((_R, 1), lambda i: (i, 0)),
        ],
        out_specs=pl.BlockSpec((1, 1), lambda i: (0, 0), memory_space=pltpu.SMEM),
        out_shape=jax.ShapeDtypeStruct((1, 1), jnp.float32),
        scratch_shapes=[pltpu.SMEM((2,), jnp.float32)],
    )


@jax.jit
def kernel(seq_embs, target_seq, mask, item_emb_table, neg_items):
    scores = _sc_scores()(
        neg_items.reshape(BS * NNEG),
        target_seq.reshape(BS),
        seq_embs.reshape(BS, D),
        item_emb_table,
    ).reshape(BS, W)
    out = _tc_loss()(scores, mask.reshape(BS, 1))
    return out[0, 0]
